# baseline (speedup 1.0000x reference)
import functools

import jax
import jax.numpy as jnp
from jax.experimental import pallas as pl
from jax.experimental.pallas import tpu as pltpu

_BN = 192


def _vq_body(idx_ref, pq_ref, ze_ref, books_ref, prob_ref, logp_ref, zq_ref,
             cb_ref):
    del idx_ref
    bn, k = prob_ref.shape[1], prob_ref.shape[2]
    zeb = ze_ref[0]
    sel = books_ref[0]
    pq = pq_ref[0]

    a = pq * jnp.sum(zeb * zeb, axis=1, keepdims=True)

    @pl.when(pl.program_id(1) == 0)
    def _():
        selT = sel.T
        cb_ref[0] = pq * jnp.sum(selT * selT, axis=0)

    cb = cb_ref[0][None, :]
    zb2 = jax.lax.dot_general(
        zeb, (2.0 * pq) * sel, (((1,), (1,)), ((), ())),
        preferred_element_type=jnp.float32,
    )
    logp_ref[0] = zb2 - (a + cb)
    logits = logp_ref[0]

    m = jnp.max(logits, axis=1, keepdims=True)
    kidx16 = jnp.argmax(logits, axis=1)

    logp_ref[0] = logits - m
    prob_ref[0] = jnp.exp(logp_ref[0])
    s = jnp.sum(prob_ref[0], axis=1, keepdims=True)
    prob_ref[0] = prob_ref[0] * (1.0 / s)
    logp_ref[0] = logp_ref[0] - jnp.log(s)

    kidx = kidx16[:, None]
    kiota = jax.lax.broadcasted_iota(jnp.int32, (bn, k), 1)
    onehot = (kiota == kidx).astype(jnp.float32)
    zq_ref[0] = jax.lax.dot_general(
        onehot, sel, (((1,), (0,)), ((), ())),
        preferred_element_type=jnp.float32,
    )


def kernel(ze, c_logits, books, log_param_q, is_train):
    b, n, d = ze.shape
    c, k, _ = books.shape
    param_q = 1.0 + jnp.exp(log_param_q)
    precision_q = 0.5 / jnp.clip(param_q, 1e-10)
    idx = jnp.argmax(c_logits, axis=-1).astype(jnp.int32)
    pq_arr = jnp.reshape(precision_q, (1,)).astype(jnp.float32)

    grid = (b, n // _BN)
    prob, logp, zq = pl.pallas_call(
        _vq_body,
        grid_spec=pltpu.PrefetchScalarGridSpec(
            num_scalar_prefetch=2,
            grid=grid,
            in_specs=[
                pl.BlockSpec((1, _BN, d), lambda i, j, idx_r, pq_r: (i, j, 0)),
                pl.BlockSpec((1, k, d), lambda i, j, idx_r, pq_r: (idx_r[i], 0, 0)),
            ],
            out_specs=[
                pl.BlockSpec((1, _BN, k), lambda i, j, idx_r, pq_r: (i, j, 0)),
                pl.BlockSpec((1, _BN, k), lambda i, j, idx_r, pq_r: (i, j, 0)),
                pl.BlockSpec((1, _BN, d), lambda i, j, idx_r, pq_r: (i, j, 0)),
            ],
            scratch_shapes=[pltpu.VMEM((1, k), jnp.float32)],
        ),
        out_shape=[
            jax.ShapeDtypeStruct((b, n, k), jnp.float32),
            jax.ShapeDtypeStruct((b, n, k), jnp.float32),
            jax.ShapeDtypeStruct((b, n, d), jnp.float32),
        ],
    )(idx, pq_arr, ze, books)
    return (zq, precision_q, prob, logp)


# speedup vs baseline: 1.0173x; 1.0173x over previous
import functools

import jax
import jax.numpy as jnp
from jax.experimental import pallas as pl
from jax.experimental.pallas import tpu as pltpu

_BN = 192


def _vq_body(idx_ref, pq_ref, ze_ref, books_ref, prob_ref, logp_ref, zq_ref,
             cb_ref):
    del idx_ref
    bn, k = prob_ref.shape[1], prob_ref.shape[2]
    zeb = ze_ref[0]
    sel = books_ref[0]
    pq = pq_ref[0]

    a = pq * jnp.sum(zeb * zeb, axis=1, keepdims=True)

    @pl.when(pl.program_id(1) == 0)
    def _():
        selT = sel.T
        cb_ref[0] = pq * jnp.sum(selT * selT, axis=0)

    cb = cb_ref[0][None, :]
    zb2 = jax.lax.dot_general(
        zeb, (2.0 * pq) * sel, (((1,), (1,)), ((), ())),
        preferred_element_type=jnp.float32,
    )
    logits = zb2 - (a + cb)

    m = jnp.max(logits, axis=1, keepdims=True)
    logp_ref[0] = logits - m
    prob_ref[0] = jnp.exp(logp_ref[0])
    s = jnp.sum(prob_ref[0], axis=1, keepdims=True)
    prob_ref[0] = prob_ref[0] * (1.0 / s)
    logp_ref[0] = logp_ref[0] - jnp.log(s)

    kidx = jnp.argmax(logits, axis=1)[:, None]
    kiota = jax.lax.broadcasted_iota(jnp.int32, (bn, k), 1)
    onehot = (kiota == kidx).astype(jnp.float32)
    zq_ref[0] = jax.lax.dot_general(
        onehot, sel, (((1,), (0,)), ((), ())),
        preferred_element_type=jnp.float32,
    )


def kernel(ze, c_logits, books, log_param_q, is_train):
    b, n, d = ze.shape
    c, k, _ = books.shape
    param_q = 1.0 + jnp.exp(log_param_q)
    precision_q = 0.5 / jnp.clip(param_q, 1e-10)
    idx = jnp.argmax(c_logits, axis=-1).astype(jnp.int32)
    pq_arr = jnp.reshape(precision_q, (1,)).astype(jnp.float32)

    grid = (b, n // _BN)
    prob, logp, zq = pl.pallas_call(
        _vq_body,
        grid_spec=pltpu.PrefetchScalarGridSpec(
            num_scalar_prefetch=2,
            grid=grid,
            in_specs=[
                pl.BlockSpec((1, _BN, d), lambda i, j, idx_r, pq_r: (i, j, 0)),
                pl.BlockSpec((1, k, d), lambda i, j, idx_r, pq_r: (idx_r[i], 0, 0)),
            ],
            out_specs=[
                pl.BlockSpec((1, _BN, k), lambda i, j, idx_r, pq_r: (i, j, 0)),
                pl.BlockSpec((1, _BN, k), lambda i, j, idx_r, pq_r: (i, j, 0)),
                pl.BlockSpec((1, _BN, d), lambda i, j, idx_r, pq_r: (i, j, 0)),
            ],
            scratch_shapes=[pltpu.VMEM((1, k), jnp.float32)],
        ),
        out_shape=[
            jax.ShapeDtypeStruct((b, n, k), jnp.float32),
            jax.ShapeDtypeStruct((b, n, k), jnp.float32),
            jax.ShapeDtypeStruct((b, n, d), jnp.float32),
        ],
    )(idx, pq_arr, ze, books)
    return (zq, precision_q, prob, logp)
